# baseline (speedup 1.0000x reference)
import functools

import jax
import jax.numpy as jnp
from jax.experimental import pallas as pl
from jax.experimental.pallas import tpu as pltpu

VMEM_LIMIT = 100 * 1024 * 1024


def _ru(x, m):
    return ((x + m - 1) // m) * m


def _sep_unit(slab, dw_ref, pw_ref, s_ref, b_ref, Wp, R8):
    B, S, Cin = slab.shape
    wv = dw_ref[...].astype(jnp.float32)
    acc = None
    for dy in range(3):
        for dx in range(3):
            off = dy * Wp + dx
            t = slab[:, off:off + R8, :].astype(jnp.float32)
            term = t * wv[dy, dx:dx + 1, :]
            acc = term if acc is None else acc + term
    a = acc.astype(jnp.bfloat16).reshape(B * R8, Cin)
    z = jnp.dot(a, pw_ref[...], preferred_element_type=jnp.float32)
    return z * s_ref[...] + b_ref[...]


def _stage(slab, z, Wp, W, R):
    B, S, C = slab.shape
    z3 = z.reshape(B, -1, C)[:, :R, :].astype(jnp.bfloat16)
    ridx = jax.lax.broadcasted_iota(jnp.int32, (1, R, 1), 1)
    zc = jnp.where((ridx % Wp) < W, z3, jnp.bfloat16(0.0))
    slab[:, Wp + 1:Wp + 1 + R, :] = zc


def _conv12_kernel(a_ref, w1_ref, s1_ref, b1_ref, w2_ref, s2_ref, b2_ref,
                   o_ref, slab):
    B, S, K = a_ref.shape
    W1 = 31
    R, R8 = 897, 904
    a = a_ref[...].reshape(B * S, K)
    y = jnp.dot(a, w1_ref[...], preferred_element_type=jnp.float32)
    y = jnp.maximum(y * s1_ref[...] + b1_ref[...], 0.0)
    slab[...] = y.reshape(B, S, 128).astype(jnp.bfloat16)
    acc = None
    for dy in range(3):
        for dx in range(3):
            off = dy * W1 + dx
            t = slab[:, off:off + R8, :]
            d = jnp.dot(t.reshape(B * R8, 128), w2_ref[3 * dy + dx],
                        preferred_element_type=jnp.float32)
            acc = d if acc is None else acc + d
    z = jnp.maximum(acc * s2_ref[...] + b2_ref[...], 0.0)
    z = z.reshape(B, R8, 128).astype(jnp.bfloat16)
    for h in range(29):
        o_ref[:, h] = z[:, h * W1:h * W1 + 29, :]


def _conv12(a, w1, s1, b1, w2, s2, b2, N, B):
    G = N // B
    return pl.pallas_call(
        _conv12_kernel,
        out_shape=jax.ShapeDtypeStruct((N, 29, 29, 128), jnp.bfloat16),
        grid=(G,),
        in_specs=[
            pl.BlockSpec((B, 968, 48), lambda i: (i, 0, 0)),
            pl.BlockSpec((48, 128), lambda i: (0, 0)),
            pl.BlockSpec((1, 128), lambda i: (0, 0)),
            pl.BlockSpec((1, 128), lambda i: (0, 0)),
            pl.BlockSpec((9, 128, 128), lambda i: (0, 0, 0)),
            pl.BlockSpec((1, 128), lambda i: (0, 0)),
            pl.BlockSpec((1, 128), lambda i: (0, 0)),
        ],
        out_specs=pl.BlockSpec((B, 29, 29, 128), lambda i: (i, 0, 0, 0)),
        scratch_shapes=[pltpu.VMEM((B, 968, 128), jnp.bfloat16)],
        compiler_params=pltpu.CompilerParams(
            dimension_semantics=("arbitrary",),
            vmem_limit_bytes=VMEM_LIMIT),
    )(a, w1, s1, b1, w2, s2, b2)


def _block_kernel(x_ref, dw0, pw0, s0, b0, dw1, pw1, s1, b1, skw, sks, skb,
                  o_ref, slab0, slab1, pslab, ximg, skm, pflat,
                  *, H, relu_in):
    B = x_ref.shape[0]
    W = H
    Wo = (W - 1) // 2 + 1
    Ho = Wo
    Wp = _ru(max(W + 2, 2 * Wo + 2), 8)
    Hpe = 2 * ((2 * Ho + 3) // 2)
    R = (H - 1) * Wp + W
    R8 = _ru(R, 8)
    Ssk = pflat.shape[1]
    Cout = o_ref.shape[3]
    NEG = jnp.bfloat16(-jnp.inf)

    slab0[...] = jnp.zeros_like(slab0)
    for h in range(H):
        row = x_ref[:, h]
        if relu_in:
            row = jnp.maximum(row, 0)
        slab0[:, (h + 1) * Wp + 1:(h + 1) * Wp + 1 + W, :] = row

    z0 = _sep_unit(slab0, dw0, pw0, s0, b0, Wp, R8)
    z0 = jnp.maximum(z0, 0.0)
    slab1[...] = jnp.zeros_like(slab1)
    _stage(slab1, z0, Wp, W, R)
    z1 = _sep_unit(slab1, dw1, pw1, s1, b1, Wp, R8)
    z1 = z1.reshape(B, R8, Cout)[:, :R, :].astype(jnp.bfloat16)

    pslab[:, :Wp + 1, :] = jnp.full((B, Wp + 1, Cout), NEG)
    pslab[:, Wp + 1 + R:, :] = jnp.full(
        (B, pslab.shape[1] - (Wp + 1 + R), Cout), NEG)
    ridx = jax.lax.broadcasted_iota(jnp.int32, (1, R, 1), 1)
    pslab[:, Wp + 1:Wp + 1 + R, :] = jnp.where((ridx % Wp) < W, z1, NEG)
    P = pslab[:, :Hpe * Wp, :].reshape(B, Hpe, Wp // 2, 2, Cout)
    xe = P[:, :, :, 0, :]
    xo = P[:, :, :, 1, :]
    wmax = jnp.maximum(jnp.maximum(xe[:, :, :Wo], xo[:, :, :Wo]),
                       xe[:, :, 1:Wo + 1])
    rh = wmax.reshape(B, Hpe // 2, 2, Wo, Cout)
    pooled = jnp.maximum(jnp.maximum(rh[:, :Ho, 0], rh[:, :Ho, 1]),
                         rh[:, 1:Ho + 1, 0])
    for ho in range(Ho):
        pflat[:, ho * Wo:(ho + 1) * Wo, :] = pooled[:, ho]

    for ho in range(Ho):
        ximg[:, ho, :W, :] = x_ref[:, 2 * ho]
    ev = ximg[...].reshape(B, Ho, Wo, 2, ximg.shape[3])[:, :, :, 0, :]
    for ho in range(Ho):
        skm[:, ho * Wo:(ho + 1) * Wo, :] = ev[:, ho]
    Cin = skm.shape[2]
    sk = jnp.dot(skm[...].reshape(B * Ssk, Cin), skw[...],
                 preferred_element_type=jnp.float32)
    sk = sk * sks[...] + skb[...]
    out = sk + pflat[...].reshape(B * Ssk, Cout).astype(jnp.float32)
    ob = out.astype(jnp.bfloat16).reshape(B, Ssk, Cout)
    for ho in range(Ho):
        o_ref[:, ho] = ob[:, ho * Wo:(ho + 1) * Wo, :]


def _block(x, u0, u1, skip, H, relu_in, N, B):
    G = N // B
    Cin = x.shape[3]
    Cmid = u0[1].shape[1]
    Cout = u1[1].shape[1]
    W = H
    Ho = (H - 1) // 2 + 1
    Wp = _ru(max(W + 2, 2 * Ho + 2), 8)
    Hpe = 2 * Ho + 2
    R = (H - 1) * Wp + W
    R8 = _ru(R, 8)
    S = _ru(2 * Wp + 2 + R8, 8)
    Sp = _ru(max(Hpe * Wp, Wp + 1 + R), 8)
    Ssk = _ru(Ho * Ho, 8)
    kern = functools.partial(_block_kernel, H=H, relu_in=relu_in)
    return pl.pallas_call(
        kern,
        out_shape=jax.ShapeDtypeStruct((N, Ho, Ho, Cout), jnp.bfloat16),
        grid=(G,),
        in_specs=[
            pl.BlockSpec((B, H, W, Cin), lambda i: (i, 0, 0, 0)),
            pl.BlockSpec((3, 3, Cin), lambda i: (0, 0, 0)),
            pl.BlockSpec((Cin, Cmid), lambda i: (0, 0)),
            pl.BlockSpec((1, Cmid), lambda i: (0, 0)),
            pl.BlockSpec((1, Cmid), lambda i: (0, 0)),
            pl.BlockSpec((3, 3, Cmid), lambda i: (0, 0, 0)),
            pl.BlockSpec((Cmid, Cout), lambda i: (0, 0)),
            pl.BlockSpec((1, Cout), lambda i: (0, 0)),
            pl.BlockSpec((1, Cout), lambda i: (0, 0)),
            pl.BlockSpec((Cin, Cout), lambda i: (0, 0)),
            pl.BlockSpec((1, Cout), lambda i: (0, 0)),
            pl.BlockSpec((1, Cout), lambda i: (0, 0)),
        ],
        out_specs=pl.BlockSpec((B, Ho, Ho, Cout), lambda i: (i, 0, 0, 0)),
        scratch_shapes=[
            pltpu.VMEM((B, S, Cin), jnp.bfloat16),
            pltpu.VMEM((B, S, Cmid), jnp.bfloat16),
            pltpu.VMEM((B, Sp, Cout), jnp.bfloat16),
            pltpu.VMEM((B, Ho, 2 * Ho, Cin), jnp.bfloat16),
            pltpu.VMEM((B, Ssk, Cin), jnp.bfloat16),
            pltpu.VMEM((B, Ssk, Cout), jnp.bfloat16),
        ],
        compiler_params=pltpu.CompilerParams(
            dimension_semantics=("arbitrary",),
            vmem_limit_bytes=VMEM_LIMIT),
    )(x, u0[0], u0[1], u0[2], u0[3], u1[0], u1[1], u1[2], u1[3],
      skip[0], skip[1], skip[2])


def _mid_kernel(x_ref, *refs):
    o_ref = refs[96]
    cur, tmp = refs[97], refs[98]
    B = x_ref.shape[0]
    H = W = 4
    Wp = 6
    R = (H - 1) * Wp + W
    R8 = _ru(R, 8)
    C = 768

    cur[...] = jnp.zeros_like(cur)
    for h in range(H):
        cur[:, (h + 1) * Wp + 1:(h + 1) * Wp + 1 + W, :] = x_ref[:, h]

    for blk in range(8):
        w = refs[12 * blk:12 * blk + 12]
        tmp[...] = jnp.maximum(cur[...], 0)
        z = _sep_unit(tmp, w[0], w[1], w[2], w[3], Wp, R8)
        _stage(tmp, jnp.maximum(z, 0.0), Wp, W, R)
        z = _sep_unit(tmp, w[4], w[5], w[6], w[7], Wp, R8)
        _stage(tmp, jnp.maximum(z, 0.0), Wp, W, R)
        z = _sep_unit(tmp, w[8], w[9], w[10], w[11], Wp, R8)
        res = cur[:, Wp + 1:Wp + 1 + R, :].astype(jnp.float32)
        y = z.reshape(B, R8, C)[:, :R, :] + res
        ridx = jax.lax.broadcasted_iota(jnp.int32, (1, R, 1), 1)
        y = jnp.where((ridx % Wp) < W, y, 0.0)
        cur[:, Wp + 1:Wp + 1 + R, :] = y.astype(jnp.bfloat16)

    for h in range(H):
        o_ref[:, h] = cur[:, (h + 1) * Wp + 1:(h + 1) * Wp + 1 + W, :]


def _mid(x, unit_params, N, B):
    G = N // B
    S = 40
    in_specs = [pl.BlockSpec((B, 4, 4, 768), lambda i: (i, 0, 0, 0))]
    args = [x]
    for (dw, pw, sc, bi) in unit_params:
        in_specs += [
            pl.BlockSpec((3, 3, 768), lambda i: (0, 0, 0)),
            pl.BlockSpec((768, 768), lambda i: (0, 0)),
            pl.BlockSpec((1, 768), lambda i: (0, 0)),
            pl.BlockSpec((1, 768), lambda i: (0, 0)),
        ]
        args += [dw, pw, sc, bi]
    return pl.pallas_call(
        _mid_kernel,
        out_shape=jax.ShapeDtypeStruct((N, 4, 4, 768), jnp.bfloat16),
        grid=(G,),
        in_specs=in_specs,
        out_specs=pl.BlockSpec((B, 4, 4, 768), lambda i: (i, 0, 0, 0)),
        scratch_shapes=[pltpu.VMEM((B, S, 768), jnp.bfloat16),
                        pltpu.VMEM((B, S, 768), jnp.bfloat16)],
        compiler_params=pltpu.CompilerParams(
            dimension_semantics=("arbitrary",),
            vmem_limit_bytes=VMEM_LIMIT),
    )(*args)


def _tail_kernel(x_ref, dw3, pw3, s3, b3, dw4, pw4, s4, b4, fw, fb,
                 o_ref, slab3, slab4):
    B = x_ref.shape[0]
    H = W = 2
    Wp = 4
    R = (H - 1) * Wp + W
    R8 = 8

    slab3[...] = jnp.zeros_like(slab3)
    for h in range(H):
        slab3[:, (h + 1) * Wp + 1:(h + 1) * Wp + 1 + W, :] = x_ref[:, h]
    z3 = _sep_unit(slab3, dw3, pw3, s3, b3, Wp, R8)
    slab4[...] = jnp.zeros_like(slab4)
    _stage(slab4, jnp.maximum(z3, 0.0), Wp, W, R)
    z4 = _sep_unit(slab4, dw4, pw4, s4, b4, Wp, R8)
    C4 = z4.shape[1]
    h4 = jnp.maximum(z4.astype(jnp.bfloat16).astype(jnp.float32), 0.0)
    h4 = h4.reshape(B, R8, C4)
    pooled = (h4[:, 0] + h4[:, 1] + h4[:, 4] + h4[:, 5]) * 0.25
    o_ref[...] = (jnp.dot(pooled, fw[...], preferred_element_type=jnp.float32)
                  + fb[...])


def _tail(x, c3, c4, fw, fb, N, B):
    G = N // B
    return pl.pallas_call(
        _tail_kernel,
        out_shape=jax.ShapeDtypeStruct((N, 128), jnp.float32),
        grid=(G,),
        in_specs=[
            pl.BlockSpec((B, 2, 2, 1024), lambda i: (i, 0, 0, 0)),
            pl.BlockSpec((3, 3, 1024), lambda i: (0, 0, 0)),
            pl.BlockSpec((1024, 1536), lambda i: (0, 0)),
            pl.BlockSpec((1, 1536), lambda i: (0, 0)),
            pl.BlockSpec((1, 1536), lambda i: (0, 0)),
            pl.BlockSpec((3, 3, 1536), lambda i: (0, 0, 0)),
            pl.BlockSpec((1536, 2048), lambda i: (0, 0)),
            pl.BlockSpec((1, 2048), lambda i: (0, 0)),
            pl.BlockSpec((1, 2048), lambda i: (0, 0)),
            pl.BlockSpec((2048, 128), lambda i: (0, 0)),
            pl.BlockSpec((1, 128), lambda i: (0, 0)),
        ],
        out_specs=pl.BlockSpec((B, 128), lambda i: (i, 0)),
        scratch_shapes=[pltpu.VMEM((B, 24, 1024), jnp.bfloat16),
                        pltpu.VMEM((B, 24, 1536), jnp.bfloat16)],
        compiler_params=pltpu.CompilerParams(
            dimension_semantics=("arbitrary",),
            vmem_limit_bytes=VMEM_LIMIT),
    )(x, c3[0], c3[1], c3[2], c3[3], c4[0], c4[1], c4[2], c4[3], fw, fb)


def kernel(x, conv1_w, bn1_scale, bn1_bias, conv2_w, bn2_scale, bn2_bias, b0_u0_dw, b0_u0_pw, b0_u0_scale, b0_u0_bias, b0_u1_dw, b0_u1_pw, b0_u1_scale, b0_u1_bias, b0_skip, b0_skip_scale, b0_skip_bias, b1_u0_dw, b1_u0_pw, b1_u0_scale, b1_u0_bias, b1_u1_dw, b1_u1_pw, b1_u1_scale, b1_u1_bias, b1_skip, b1_skip_scale, b1_skip_bias, b2_u0_dw, b2_u0_pw, b2_u0_scale, b2_u0_bias, b2_u1_dw, b2_u1_pw, b2_u1_scale, b2_u1_bias, b2_skip, b2_skip_scale, b2_skip_bias, b3_u0_dw, b3_u0_pw, b3_u0_scale, b3_u0_bias, b3_u1_dw, b3_u1_pw, b3_u1_scale, b3_u1_bias, b3_u2_dw, b3_u2_pw, b3_u2_scale, b3_u2_bias, b4_u0_dw, b4_u0_pw, b4_u0_scale, b4_u0_bias, b4_u1_dw, b4_u1_pw, b4_u1_scale, b4_u1_bias, b4_u2_dw, b4_u2_pw, b4_u2_scale, b4_u2_bias, b5_u0_dw, b5_u0_pw, b5_u0_scale, b5_u0_bias, b5_u1_dw, b5_u1_pw, b5_u1_scale, b5_u1_bias, b5_u2_dw, b5_u2_pw, b5_u2_scale, b5_u2_bias, b6_u0_dw, b6_u0_pw, b6_u0_scale, b6_u0_bias, b6_u1_dw, b6_u1_pw, b6_u1_scale, b6_u1_bias, b6_u2_dw, b6_u2_pw, b6_u2_scale, b6_u2_bias, b7_u0_dw, b7_u0_pw, b7_u0_scale, b7_u0_bias, b7_u1_dw, b7_u1_pw, b7_u1_scale, b7_u1_bias, b7_u2_dw, b7_u2_pw, b7_u2_scale, b7_u2_bias, b8_u0_dw, b8_u0_pw, b8_u0_scale, b8_u0_bias, b8_u1_dw, b8_u1_pw, b8_u1_scale, b8_u1_bias, b8_u2_dw, b8_u2_pw, b8_u2_scale, b8_u2_bias, b9_u0_dw, b9_u0_pw, b9_u0_scale, b9_u0_bias, b9_u1_dw, b9_u1_pw, b9_u1_scale, b9_u1_bias, b9_u2_dw, b9_u2_pw, b9_u2_scale, b9_u2_bias, b10_u0_dw, b10_u0_pw, b10_u0_scale, b10_u0_bias, b10_u1_dw, b10_u1_pw, b10_u1_scale, b10_u1_bias, b10_u2_dw, b10_u2_pw, b10_u2_scale, b10_u2_bias, b11_u0_dw, b11_u0_pw, b11_u0_scale, b11_u0_bias, b11_u1_dw, b11_u1_pw, b11_u1_scale, b11_u1_bias, b11_skip, b11_skip_scale, b11_skip_bias, conv3_dw, conv3_pw, conv3_scale, conv3_bias, conv4_dw, conv4_pw, conv4_scale, conv4_bias, fc_w, fc_b):
    L = locals()
    N = x.shape[0]
    B = N

    xh = jnp.transpose(x, (0, 2, 3, 1)).astype(jnp.bfloat16)
    P = xh.reshape(N, 32, 2, 32, 2, 3).transpose(0, 1, 3, 2, 4, 5)
    P = P.reshape(N, 32, 32, 12)
    parts = [P[:, di:di + 31, dj:dj + 31, :]
             for di in range(2) for dj in range(2)]
    a = jnp.concatenate(parts, axis=-1).reshape(N, 961, 48)
    a = jnp.pad(a, ((0, 0), (0, 7), (0, 0)))

    idx = []
    msk = []
    for di in range(2):
        for dj in range(2):
            for a2 in range(2):
                for b2 in range(2):
                    for c in range(3):
                        ky, kx = 2 * di + a2, 2 * dj + b2
                        ok = (ky < 3) and (kx < 3)
                        idx.append((ky * 3 + kx) * 3 + c if ok else 0)
                        msk.append(1.0 if ok else 0.0)
    w48 = conv1_w[jnp.array(idx)] * jnp.array(msk, jnp.bfloat16)[:, None]

    y = _conv12(a, w48, bn1_scale, bn1_bias,
                conv2_w, bn2_scale, bn2_bias, N, B)

    for bi, H in ((0, 29), (1, 15), (2, 8)):
        u0 = (L[f"b{bi}_u0_dw"], L[f"b{bi}_u0_pw"],
              L[f"b{bi}_u0_scale"], L[f"b{bi}_u0_bias"])
        u1 = (L[f"b{bi}_u1_dw"], L[f"b{bi}_u1_pw"],
              L[f"b{bi}_u1_scale"], L[f"b{bi}_u1_bias"])
        sk = (L[f"b{bi}_skip"], L[f"b{bi}_skip_scale"], L[f"b{bi}_skip_bias"])
        y = _block(y, u0, u1, sk, H, relu_in=(bi != 0), N=N, B=B)

    units = []
    for bi in range(3, 11):
        for ui in range(3):
            units.append((L[f"b{bi}_u{ui}_dw"], L[f"b{bi}_u{ui}_pw"],
                          L[f"b{bi}_u{ui}_scale"], L[f"b{bi}_u{ui}_bias"]))
    y = _mid(y, units, N, B)

    u0 = (b11_u0_dw, b11_u0_pw, b11_u0_scale, b11_u0_bias)
    u1 = (b11_u1_dw, b11_u1_pw, b11_u1_scale, b11_u1_bias)
    sk = (b11_skip, b11_skip_scale, b11_skip_bias)
    y = _block(y, u0, u1, sk, 4, relu_in=True, N=N, B=B)

    fw = jnp.pad(fc_w, ((0, 0), (0, 126)))
    fb = jnp.pad(fc_b, (0, 126)).reshape(1, 128)
    out = _tail(y, (conv3_dw, conv3_pw, conv3_scale, conv3_bias),
                (conv4_dw, conv4_pw, conv4_scale, conv4_bias), fw, fb, N, B)
    return out[:, :2]


# speedup vs baseline: 1.0294x; 1.0294x over previous
import functools

import jax
import jax.numpy as jnp
from jax.experimental import pallas as pl
from jax.experimental.pallas import tpu as pltpu

VMEM_LIMIT = 100 * 1024 * 1024


def _ru(x, m):
    return ((x + m - 1) // m) * m


def _sep_unit(slab, dw_ref, pw_ref, s_ref, b_ref, Wp, R8):
    B, S, Cin = slab.shape
    wv = dw_ref[...].astype(jnp.float32)
    acc = None
    for dy in range(3):
        for dx in range(3):
            off = dy * Wp + dx
            t = slab[:, off:off + R8, :].astype(jnp.float32)
            term = t * wv[dy, dx:dx + 1, :]
            acc = term if acc is None else acc + term
    a = acc.astype(jnp.bfloat16).reshape(B * R8, Cin)
    z = jnp.dot(a, pw_ref[...], preferred_element_type=jnp.float32)
    return z * s_ref[...] + b_ref[...]


def _stage(slab, z, Wp, W, R):
    B, S, C = slab.shape
    z3 = z.reshape(B, -1, C)[:, :R, :].astype(jnp.bfloat16)
    ridx = jax.lax.broadcasted_iota(jnp.int32, (1, R, 1), 1)
    zc = jnp.where((ridx % Wp) < W, z3, jnp.bfloat16(0.0))
    slab[:, Wp + 1:Wp + 1 + R, :] = zc


def _conv12_kernel(a_ref, w1_ref, s1_ref, b1_ref, w2_ref, s2_ref, b2_ref,
                   o_ref, slab):
    B, S, K = a_ref.shape
    W1 = 31
    R, R8 = 897, 904
    a = a_ref[...].reshape(B * S, K)
    y = jnp.dot(a, w1_ref[...], preferred_element_type=jnp.float32)
    y = jnp.maximum(y * s1_ref[...] + b1_ref[...], 0.0)
    slab[...] = y.reshape(B, S, 128).astype(jnp.bfloat16)
    acc = None
    for dy in range(3):
        for dx in range(3):
            off = dy * W1 + dx
            t = slab[:, off:off + R8, :]
            d = jnp.dot(t.reshape(B * R8, 128), w2_ref[3 * dy + dx],
                        preferred_element_type=jnp.float32)
            acc = d if acc is None else acc + d
    z = jnp.maximum(acc * s2_ref[...] + b2_ref[...], 0.0)
    z = z.reshape(B, R8, 128).astype(jnp.bfloat16)
    for h in range(29):
        o_ref[:, h] = z[:, h * W1:h * W1 + 29, :]


def _conv12(a, w1, s1, b1, w2, s2, b2, N, B):
    G = N // B
    return pl.pallas_call(
        _conv12_kernel,
        out_shape=jax.ShapeDtypeStruct((N, 29, 29, 128), jnp.bfloat16),
        grid=(G,),
        in_specs=[
            pl.BlockSpec((B, 968, 48), lambda i: (i, 0, 0)),
            pl.BlockSpec((48, 128), lambda i: (0, 0)),
            pl.BlockSpec((1, 128), lambda i: (0, 0)),
            pl.BlockSpec((1, 128), lambda i: (0, 0)),
            pl.BlockSpec((9, 128, 128), lambda i: (0, 0, 0)),
            pl.BlockSpec((1, 128), lambda i: (0, 0)),
            pl.BlockSpec((1, 128), lambda i: (0, 0)),
        ],
        out_specs=pl.BlockSpec((B, 29, 29, 128), lambda i: (i, 0, 0, 0)),
        scratch_shapes=[pltpu.VMEM((B, 968, 128), jnp.bfloat16)],
        compiler_params=pltpu.CompilerParams(
            dimension_semantics=("arbitrary",),
            vmem_limit_bytes=VMEM_LIMIT),
    )(a, w1, s1, b1, w2, s2, b2)


def _block_kernel(x_ref, dw0, pw0, s0, b0, dw1, pw1, s1, b1, skw, sks, skb,
                  o_ref, slab0, slab1, pslab, ximg, skm, pflat,
                  *, H, relu_in):
    B = x_ref.shape[0]
    W = H
    Wp = W + 2
    R = (H - 1) * Wp + W
    R8 = _ru(R, 8)
    Ho = (H - 1) // 2 + 1
    Wo = Ho
    Ssk = pflat.shape[1]
    Cout = o_ref.shape[3]

    slab0[...] = jnp.zeros_like(slab0)
    for h in range(H):
        row = x_ref[:, h]
        if relu_in:
            row = jnp.maximum(row, 0)
        slab0[:, (h + 1) * Wp + 1:(h + 1) * Wp + 1 + W, :] = row

    z0 = _sep_unit(slab0, dw0, pw0, s0, b0, Wp, R8)
    z0 = jnp.maximum(z0, 0.0)
    slab1[...] = jnp.zeros_like(slab1)
    _stage(slab1, z0, Wp, W, R)
    z1 = _sep_unit(slab1, dw1, pw1, s1, b1, Wp, R8)
    z1 = z1.reshape(B, R8, Cout)[:, :R, :].astype(jnp.bfloat16)

    pslab[...] = jnp.full(pslab.shape, -jnp.inf, pslab.dtype)
    for h in range(H):
        pslab[:, h + 1, 1:W + 1, :] = z1[:, h * Wp:h * Wp + W, :]
    P = pslab[...].reshape(B, 2 * Ho + 2, Wo + 1, 2, Cout)
    xe = P[:, :, :, 0, :]
    xo = P[:, :, :, 1, :]
    wmax = jnp.maximum(jnp.maximum(xe[:, :, :Wo], xo[:, :, :Wo]),
                       xe[:, :, 1:Wo + 1])
    rh = wmax.reshape(B, Ho + 1, 2, Wo, Cout)
    pooled = jnp.maximum(jnp.maximum(rh[:, :Ho, 0], rh[:, :Ho, 1]),
                         rh[:, 1:Ho + 1, 0])
    for ho in range(Ho):
        pflat[:, ho * Wo:(ho + 1) * Wo, :] = pooled[:, ho]

    for ho in range(Ho):
        ximg[:, ho, :W, :] = x_ref[:, 2 * ho]
    ev = ximg[...].reshape(B, Ho, Wo, 2, ximg.shape[3])[:, :, :, 0, :]
    for ho in range(Ho):
        skm[:, ho * Wo:(ho + 1) * Wo, :] = ev[:, ho]
    Cin = skm.shape[2]
    sk = jnp.dot(skm[...].reshape(B * Ssk, Cin), skw[...],
                 preferred_element_type=jnp.float32)
    sk = sk * sks[...] + skb[...]
    out = sk + pflat[...].reshape(B * Ssk, Cout).astype(jnp.float32)
    ob = out.astype(jnp.bfloat16).reshape(B, Ssk, Cout)
    for ho in range(Ho):
        o_ref[:, ho] = ob[:, ho * Wo:(ho + 1) * Wo, :]


def _block(x, u0, u1, skip, H, relu_in, N, B):
    G = N // B
    Cin = x.shape[3]
    Cmid = u0[1].shape[1]
    Cout = u1[1].shape[1]
    W = H
    Ho = (H - 1) // 2 + 1
    Wp = W + 2
    R8 = _ru((H - 1) * Wp + W, 8)
    S = _ru(2 * Wp + 2 + R8, 8)
    Ssk = _ru(Ho * Ho, 8)
    kern = functools.partial(_block_kernel, H=H, relu_in=relu_in)
    return pl.pallas_call(
        kern,
        out_shape=jax.ShapeDtypeStruct((N, Ho, Ho, Cout), jnp.bfloat16),
        grid=(G,),
        in_specs=[
            pl.BlockSpec((B, H, W, Cin), lambda i: (i, 0, 0, 0)),
            pl.BlockSpec((3, 3, Cin), lambda i: (0, 0, 0)),
            pl.BlockSpec((Cin, Cmid), lambda i: (0, 0)),
            pl.BlockSpec((1, Cmid), lambda i: (0, 0)),
            pl.BlockSpec((1, Cmid), lambda i: (0, 0)),
            pl.BlockSpec((3, 3, Cmid), lambda i: (0, 0, 0)),
            pl.BlockSpec((Cmid, Cout), lambda i: (0, 0)),
            pl.BlockSpec((1, Cout), lambda i: (0, 0)),
            pl.BlockSpec((1, Cout), lambda i: (0, 0)),
            pl.BlockSpec((Cin, Cout), lambda i: (0, 0)),
            pl.BlockSpec((1, Cout), lambda i: (0, 0)),
            pl.BlockSpec((1, Cout), lambda i: (0, 0)),
        ],
        out_specs=pl.BlockSpec((B, Ho, Ho, Cout), lambda i: (i, 0, 0, 0)),
        scratch_shapes=[
            pltpu.VMEM((B, S, Cin), jnp.bfloat16),
            pltpu.VMEM((B, S, Cmid), jnp.bfloat16),
            pltpu.VMEM((B, 2 * Ho + 2, 2 * Ho + 2, Cout), jnp.bfloat16),
            pltpu.VMEM((B, Ho, 2 * Ho, Cin), jnp.bfloat16),
            pltpu.VMEM((B, Ssk, Cin), jnp.bfloat16),
            pltpu.VMEM((B, Ssk, Cout), jnp.bfloat16),
        ],
        compiler_params=pltpu.CompilerParams(
            dimension_semantics=("arbitrary",),
            vmem_limit_bytes=VMEM_LIMIT),
    )(x, u0[0], u0[1], u0[2], u0[3], u1[0], u1[1], u1[2], u1[3],
      skip[0], skip[1], skip[2])


def _mid_kernel(x_ref, *refs):
    o_ref = refs[96]
    cur, tmp = refs[97], refs[98]
    B = x_ref.shape[0]
    H = W = 4
    Wp = 6
    R = (H - 1) * Wp + W
    R8 = _ru(R, 8)
    C = 768

    cur[...] = jnp.zeros_like(cur)
    for h in range(H):
        cur[:, (h + 1) * Wp + 1:(h + 1) * Wp + 1 + W, :] = x_ref[:, h]

    for blk in range(8):
        w = refs[12 * blk:12 * blk + 12]
        tmp[...] = jnp.maximum(cur[...], 0)
        z = _sep_unit(tmp, w[0], w[1], w[2], w[3], Wp, R8)
        _stage(tmp, jnp.maximum(z, 0.0), Wp, W, R)
        z = _sep_unit(tmp, w[4], w[5], w[6], w[7], Wp, R8)
        _stage(tmp, jnp.maximum(z, 0.0), Wp, W, R)
        z = _sep_unit(tmp, w[8], w[9], w[10], w[11], Wp, R8)
        res = cur[:, Wp + 1:Wp + 1 + R, :].astype(jnp.float32)
        y = z.reshape(B, R8, C)[:, :R, :] + res
        ridx = jax.lax.broadcasted_iota(jnp.int32, (1, R, 1), 1)
        y = jnp.where((ridx % Wp) < W, y, 0.0)
        cur[:, Wp + 1:Wp + 1 + R, :] = y.astype(jnp.bfloat16)

    for h in range(H):
        o_ref[:, h] = cur[:, (h + 1) * Wp + 1:(h + 1) * Wp + 1 + W, :]


def _mid(x, unit_params, N, B):
    G = N // B
    S = 40
    in_specs = [pl.BlockSpec((B, 4, 4, 768), lambda i: (i, 0, 0, 0))]
    args = [x]
    for (dw, pw, sc, bi) in unit_params:
        in_specs += [
            pl.BlockSpec((3, 3, 768), lambda i: (0, 0, 0)),
            pl.BlockSpec((768, 768), lambda i: (0, 0)),
            pl.BlockSpec((1, 768), lambda i: (0, 0)),
            pl.BlockSpec((1, 768), lambda i: (0, 0)),
        ]
        args += [dw, pw, sc, bi]
    return pl.pallas_call(
        _mid_kernel,
        out_shape=jax.ShapeDtypeStruct((N, 4, 4, 768), jnp.bfloat16),
        grid=(G,),
        in_specs=in_specs,
        out_specs=pl.BlockSpec((B, 4, 4, 768), lambda i: (i, 0, 0, 0)),
        scratch_shapes=[pltpu.VMEM((B, S, 768), jnp.bfloat16),
                        pltpu.VMEM((B, S, 768), jnp.bfloat16)],
        compiler_params=pltpu.CompilerParams(
            dimension_semantics=("arbitrary",),
            vmem_limit_bytes=VMEM_LIMIT),
    )(*args)


def _tail_kernel(x_ref, dw3, pw3, s3, b3, dw4, pw4, s4, b4, fw, fb,
                 o_ref, slab3, slab4):
    B = x_ref.shape[0]
    H = W = 2
    Wp = 4
    R = (H - 1) * Wp + W
    R8 = 8

    slab3[...] = jnp.zeros_like(slab3)
    for h in range(H):
        slab3[:, (h + 1) * Wp + 1:(h + 1) * Wp + 1 + W, :] = x_ref[:, h]
    z3 = _sep_unit(slab3, dw3, pw3, s3, b3, Wp, R8)
    slab4[...] = jnp.zeros_like(slab4)
    _stage(slab4, jnp.maximum(z3, 0.0), Wp, W, R)
    z4 = _sep_unit(slab4, dw4, pw4, s4, b4, Wp, R8)
    C4 = z4.shape[1]
    h4 = jnp.maximum(z4.astype(jnp.bfloat16).astype(jnp.float32), 0.0)
    h4 = h4.reshape(B, R8, C4)
    pooled = (h4[:, 0] + h4[:, 1] + h4[:, 4] + h4[:, 5]) * 0.25
    o_ref[...] = (jnp.dot(pooled, fw[...], preferred_element_type=jnp.float32)
                  + fb[...])


def _tail(x, c3, c4, fw, fb, N, B):
    G = N // B
    return pl.pallas_call(
        _tail_kernel,
        out_shape=jax.ShapeDtypeStruct((N, 128), jnp.float32),
        grid=(G,),
        in_specs=[
            pl.BlockSpec((B, 2, 2, 1024), lambda i: (i, 0, 0, 0)),
            pl.BlockSpec((3, 3, 1024), lambda i: (0, 0, 0)),
            pl.BlockSpec((1024, 1536), lambda i: (0, 0)),
            pl.BlockSpec((1, 1536), lambda i: (0, 0)),
            pl.BlockSpec((1, 1536), lambda i: (0, 0)),
            pl.BlockSpec((3, 3, 1536), lambda i: (0, 0, 0)),
            pl.BlockSpec((1536, 2048), lambda i: (0, 0)),
            pl.BlockSpec((1, 2048), lambda i: (0, 0)),
            pl.BlockSpec((1, 2048), lambda i: (0, 0)),
            pl.BlockSpec((2048, 128), lambda i: (0, 0)),
            pl.BlockSpec((1, 128), lambda i: (0, 0)),
        ],
        out_specs=pl.BlockSpec((B, 128), lambda i: (i, 0)),
        scratch_shapes=[pltpu.VMEM((B, 24, 1024), jnp.bfloat16),
                        pltpu.VMEM((B, 24, 1536), jnp.bfloat16)],
        compiler_params=pltpu.CompilerParams(
            dimension_semantics=("arbitrary",),
            vmem_limit_bytes=VMEM_LIMIT),
    )(x, c3[0], c3[1], c3[2], c3[3], c4[0], c4[1], c4[2], c4[3], fw, fb)


def kernel(x, conv1_w, bn1_scale, bn1_bias, conv2_w, bn2_scale, bn2_bias, b0_u0_dw, b0_u0_pw, b0_u0_scale, b0_u0_bias, b0_u1_dw, b0_u1_pw, b0_u1_scale, b0_u1_bias, b0_skip, b0_skip_scale, b0_skip_bias, b1_u0_dw, b1_u0_pw, b1_u0_scale, b1_u0_bias, b1_u1_dw, b1_u1_pw, b1_u1_scale, b1_u1_bias, b1_skip, b1_skip_scale, b1_skip_bias, b2_u0_dw, b2_u0_pw, b2_u0_scale, b2_u0_bias, b2_u1_dw, b2_u1_pw, b2_u1_scale, b2_u1_bias, b2_skip, b2_skip_scale, b2_skip_bias, b3_u0_dw, b3_u0_pw, b3_u0_scale, b3_u0_bias, b3_u1_dw, b3_u1_pw, b3_u1_scale, b3_u1_bias, b3_u2_dw, b3_u2_pw, b3_u2_scale, b3_u2_bias, b4_u0_dw, b4_u0_pw, b4_u0_scale, b4_u0_bias, b4_u1_dw, b4_u1_pw, b4_u1_scale, b4_u1_bias, b4_u2_dw, b4_u2_pw, b4_u2_scale, b4_u2_bias, b5_u0_dw, b5_u0_pw, b5_u0_scale, b5_u0_bias, b5_u1_dw, b5_u1_pw, b5_u1_scale, b5_u1_bias, b5_u2_dw, b5_u2_pw, b5_u2_scale, b5_u2_bias, b6_u0_dw, b6_u0_pw, b6_u0_scale, b6_u0_bias, b6_u1_dw, b6_u1_pw, b6_u1_scale, b6_u1_bias, b6_u2_dw, b6_u2_pw, b6_u2_scale, b6_u2_bias, b7_u0_dw, b7_u0_pw, b7_u0_scale, b7_u0_bias, b7_u1_dw, b7_u1_pw, b7_u1_scale, b7_u1_bias, b7_u2_dw, b7_u2_pw, b7_u2_scale, b7_u2_bias, b8_u0_dw, b8_u0_pw, b8_u0_scale, b8_u0_bias, b8_u1_dw, b8_u1_pw, b8_u1_scale, b8_u1_bias, b8_u2_dw, b8_u2_pw, b8_u2_scale, b8_u2_bias, b9_u0_dw, b9_u0_pw, b9_u0_scale, b9_u0_bias, b9_u1_dw, b9_u1_pw, b9_u1_scale, b9_u1_bias, b9_u2_dw, b9_u2_pw, b9_u2_scale, b9_u2_bias, b10_u0_dw, b10_u0_pw, b10_u0_scale, b10_u0_bias, b10_u1_dw, b10_u1_pw, b10_u1_scale, b10_u1_bias, b10_u2_dw, b10_u2_pw, b10_u2_scale, b10_u2_bias, b11_u0_dw, b11_u0_pw, b11_u0_scale, b11_u0_bias, b11_u1_dw, b11_u1_pw, b11_u1_scale, b11_u1_bias, b11_skip, b11_skip_scale, b11_skip_bias, conv3_dw, conv3_pw, conv3_scale, conv3_bias, conv4_dw, conv4_pw, conv4_scale, conv4_bias, fc_w, fc_b):
    L = locals()
    N = x.shape[0]
    B = N

    P = x.astype(jnp.bfloat16).reshape(N, 3, 32, 2, 32, 2)
    P = P.transpose(0, 2, 4, 3, 5, 1).reshape(N, 32, 32, 12)
    parts = [P[:, di:di + 31, dj:dj + 31, :]
             for di in range(2) for dj in range(2)]
    a = jnp.concatenate(parts, axis=-1).reshape(N, 961, 48)
    a = jnp.pad(a, ((0, 0), (0, 7), (0, 0)))

    idx = []
    msk = []
    for di in range(2):
        for dj in range(2):
            for a2 in range(2):
                for b2 in range(2):
                    for c in range(3):
                        ky, kx = 2 * di + a2, 2 * dj + b2
                        ok = (ky < 3) and (kx < 3)
                        idx.append((ky * 3 + kx) * 3 + c if ok else 0)
                        msk.append(1.0 if ok else 0.0)
    w48 = conv1_w[jnp.array(idx)] * jnp.array(msk, jnp.bfloat16)[:, None]

    y = _conv12(a, w48, bn1_scale, bn1_bias,
                conv2_w, bn2_scale, bn2_bias, N, B)

    for bi, H in ((0, 29), (1, 15), (2, 8)):
        u0 = (L[f"b{bi}_u0_dw"], L[f"b{bi}_u0_pw"],
              L[f"b{bi}_u0_scale"], L[f"b{bi}_u0_bias"])
        u1 = (L[f"b{bi}_u1_dw"], L[f"b{bi}_u1_pw"],
              L[f"b{bi}_u1_scale"], L[f"b{bi}_u1_bias"])
        sk = (L[f"b{bi}_skip"], L[f"b{bi}_skip_scale"], L[f"b{bi}_skip_bias"])
        y = _block(y, u0, u1, sk, H, relu_in=(bi != 0), N=N, B=B)

    units = []
    for bi in range(3, 11):
        for ui in range(3):
            units.append((L[f"b{bi}_u{ui}_dw"], L[f"b{bi}_u{ui}_pw"],
                          L[f"b{bi}_u{ui}_scale"], L[f"b{bi}_u{ui}_bias"]))
    y = _mid(y, units, N, B)

    u0 = (b11_u0_dw, b11_u0_pw, b11_u0_scale, b11_u0_bias)
    u1 = (b11_u1_dw, b11_u1_pw, b11_u1_scale, b11_u1_bias)
    sk = (b11_skip, b11_skip_scale, b11_skip_bias)
    y = _block(y, u0, u1, sk, 4, relu_in=True, N=N, B=B)

    fw = jnp.pad(fc_w, ((0, 0), (0, 126)))
    fb = jnp.pad(fc_b, (0, 126)).reshape(1, 128)
    out = _tail(y, (conv3_dw, conv3_pw, conv3_scale, conv3_bias),
                (conv4_dw, conv4_pw, conv4_scale, conv4_bias), fw, fb, N, B)
    return out[:, :2]
